# baseline (speedup 1.0000x reference)
import functools

import numpy as np
import jax
import jax.numpy as jnp
from jax import lax
from jax.experimental import pallas as pl
from jax.experimental.pallas import tpu as pltpu
from jax.experimental.pallas import tpu_sc as plsc

_MAX_LEN = 200
_EMB_DIM = 64
_NW = 32
_NBUF = 4
_LAG = 2
_BLKR = 16
_L0 = 104
_L1 = 96


def _make_pe_np():
    pos = np.expand_dims(np.arange(_MAX_LEN), 1)
    pe = pos / np.power(
        1000, 2 * np.expand_dims(np.arange(_EMB_DIM) // 2, 0) / _EMB_DIM
    )
    pe = pe.astype(np.float64)
    pe[:, 0::2] = np.sin(pe[:, 0::2])
    pe[:, 1::2] = np.cos(pe[:, 1::2])
    return pe.astype(np.float32)


_PE = _make_pe_np()
_HL = (_L0, _L1)
_HOFF = (0, _L0)


def _emb_sc(table2, xi, pe):
    n_rows = xi.shape[0]
    rows_per_w = n_rows // _NW
    n_chunks = rows_per_w * 2
    n_blks = rows_per_w // _BLKR
    rpb = _BLKR * 2 // _NBUF
    n_rounds = n_chunks // _NBUF
    mesh = plsc.VectorSubcoreMesh(core_axis_name="c", subcore_axis_name="s")

    @functools.partial(
        pl.kernel,
        mesh=mesh,
        out_type=jax.ShapeDtypeStruct((n_rows, _MAX_LEN, _EMB_DIM), jnp.float32),
        scratch_types=[
            pltpu.VMEM((2, _BLKR, _MAX_LEN), jnp.int32),
            pltpu.VMEM((2, _BLKR, 256), jnp.int32),
            pltpu.VMEM((_MAX_LEN, _EMB_DIM), jnp.float32),
            pltpu.VMEM((_NBUF, _L0, 2 * _EMB_DIM), jnp.float32),
            pltpu.VMEM((2, _L0, _EMB_DIM), jnp.float32),
            pltpu.SemaphoreType.DMA,
            pltpu.SemaphoreType.DMA((_NBUF,)),
            pltpu.SemaphoreType.DMA((2,)),
        ],
        compiler_params=pltpu.CompilerParams(needs_layout_passes=False),
    )
    def k(tab_h, xi_h, pe_h, out_h, idx_v, idx2_v, pe_v, rows_v,
          cbuf_v, sem_ix, sem_g, sem_o):
        cid = lax.axis_index("c")
        sid = lax.axis_index("s")
        wid = sid * 2 + cid
        base_row = wid * rows_per_w
        pltpu.sync_copy(pe_h, pe_v)

        def stage_idx(s_blk, buf, sync):
            src = xi_h.at[pl.ds(base_row + s_blk * _BLKR, _BLKR)]
            if sync:
                pltpu.sync_copy(src, idx_v.at[buf])
            else:
                pltpu.async_copy(src, idx_v.at[buf], sem_ix)

        def wait_idx(buf):
            pltpu.make_async_copy(
                xi_h.at[pl.ds(0, _BLKR)], idx_v.at[buf], sem_ix
            ).wait()

        def shift_idx(buf):
            def row_it(rr, carry):
                for h in range(2):
                    offs = (
                        (0, 16, 32, 48, 64, 80, 88)
                        if h == 0
                        else (104, 112, 128, 144, 160, 176, 184)
                    )
                    s0 = _HOFF[h]
                    d0 = h * 128
                    for so in offs:
                        idx2_v[buf, rr, pl.ds(d0 + so - s0, 16)] = idx_v[
                            buf, rr, pl.ds(so, 16)
                        ]
                return carry

            lax.fori_loop(0, _BLKR, row_it, 0)

        def start_gather(g, buf, rr, h):
            pltpu.async_copy(
                tab_h.at[idx2_v.at[buf, rr, pl.ds(h * 128, _HL[h])]],
                rows_v.at[g, pl.ds(0, _HL[h])],
                sem_g.at[g],
            )

        def wait_gather(g, h):
            pltpu.make_async_copy(
                tab_h.at[idx2_v.at[0, 0, pl.ds(0, _HL[h])]],
                rows_v.at[g, pl.ds(0, _HL[h])],
                sem_g.at[g],
            ).wait()

        def start_store(ss, row, h):
            pltpu.async_copy(
                cbuf_v.at[ss, pl.ds(0, _HL[h])],
                out_h.at[row, pl.ds(_HOFF[h], _HL[h])],
                sem_o.at[ss],
            )

        def wait_store(ss, h):
            pltpu.make_async_copy(
                cbuf_v.at[ss, pl.ds(0, _HL[h])],
                out_h.at[0, pl.ds(_HOFF[h], _HL[h])],
                sem_o.at[ss],
            ).wait()

        def compact_pe(g, ss, h):
            def row_it(i, carry):
                for j in range(4):
                    sl = pl.ds(16 * j, 16)
                    cbuf_v[ss, i, sl] = (
                        rows_v[g, i, sl] + pe_v[_HOFF[h] + i, sl]
                    )
                return carry

            lax.fori_loop(0, _HL[h], row_it, 0)

        stage_idx(0, 0, True)
        shift_idx(0)
        stage_idx(1, 1, False)

        for b in range(_NBUF):
            h = b % 2
            start_gather(b, 0, b // 2, h)
            if b >= _LAG:
                cd = b - _LAG
                wait_gather(cd % _NBUF, h)
                compact_pe(cd % _NBUF, h, h)
                start_store(h, base_row + cd // 2, h)

        def round_body(r, carry):
            s_blk = r // rpb
            buf = lax.rem(s_blk, 2)

            @pl.when(lax.rem(r, rpb) == 0)
            def _():
                wait_idx(buf)
                shift_idx(buf)

            for b in range(_NBUF):
                c = r * _NBUF + b
                h = b % 2
                row = 2 * r + b // 2
                rr = lax.rem(row, _BLKR)
                start_gather(b, buf, rr, h)
                cd_row = 2 * r + (b - 2) // 2
                sg = (b + _LAG) % _NBUF
                wait_gather(sg, h)
                wait_store(h, h)
                compact_pe(sg, h, h)
                start_store(h, base_row + cd_row, h)
                if b == _LAG - 1:
                    @pl.when((lax.rem(r, rpb) == 0) & (s_blk < n_blks - 1))
                    def _():
                        stage_idx(s_blk + 1, 1 - buf, False)
            return carry

        lax.fori_loop(1, n_rounds, round_body, 0)

        for e in range(_LAG):
            cd = n_chunks - _LAG + e
            h = cd % 2
            sg = cd % _NBUF
            wait_gather(sg, h)
            wait_store(h, h)
            compact_pe(sg, h, h)
            start_store(h, base_row + rows_per_w - 1, h)
        for h in range(2):
            wait_store(h, h)

    return k(table2, xi, pe)


def kernel(x, table):
    xi = x.astype(jnp.int32)
    table2 = jnp.pad(table, ((0, 0), (0, _EMB_DIM)))
    pe = jnp.asarray(_PE)
    return _emb_sc(table2, xi, pe)
